# baseline (speedup 1.0000x reference)
import functools

import jax
import jax.numpy as jnp
from jax import lax
from jax.experimental import pallas as pl
from jax.experimental.pallas import tpu as pltpu
from jax.experimental.pallas import tpu_sc as plsc

_NF = 26
_ND = 13
_D = 16
_NSLOT = _NF + _ND
_FIELD_SIZE = 40000
_NW = 32
_CHUNK = 128


def _body(sp_hbm, spf_hbm, dx_hbm, lin_hbm, lw_hbm, embf_hbm, ew_hbm,
          v1_hbm, v2_hbm,
          sp_v, spf_v, dx_v, ew_v, lw_v, idx2_v, lin_v, out_v, v1_v,
          sem_e, sem_l, sem_w):
    spw = spf_v.shape[1]
    nchunk = spw // _CHUNK
    wid = lax.axis_index("s") * 2 + lax.axis_index("c")

    pltpu.sync_copy(sp_hbm.at[pl.ds(wid * spw, spw)], sp_v)
    pltpu.sync_copy(spf_hbm.at[wid], spf_v)
    pltpu.sync_copy(dx_hbm.at[wid], dx_v)
    pltpu.sync_copy(ew_hbm, ew_v)
    pltpu.sync_copy(lw_hbm, lw_v)

    lanes = lax.iota(jnp.int32, 16)
    lw_reg = lw_v[pl.ds(0, 16)]
    off_lo = lanes * _FIELD_SIZE
    off_hi = (lanes + 10) * _FIELD_SIZE

    def chunk_body(c, _):
        base = wid * spw + c * _CHUNK

        @pl.when(c > 0)
        def _():
            pltpu.make_async_copy(
                out_v, v2_hbm.at[pl.ds(0, _CHUNK * _NSLOT)], sem_w).wait()
            pltpu.make_async_copy(v1_v, v1_hbm.at[pl.ds(0, _CHUNK)], sem_w).wait()

        def samp(n, _):
            row0 = n * _NSLOT
            va = (sp_v[c * _CHUNK + n, pl.ds(0, 16)] + off_lo) * _D
            vb = (sp_v[c * _CHUNK + n, pl.ds(10, 16)] + off_hi) * _D
            for k in range(16):
                pltpu.make_async_copy(
                    embf_hbm.at[va[k] + lanes], out_v.at[row0 + k],
                    sem_e).start()
            for k in range(6, 16):
                pltpu.make_async_copy(
                    embf_hbm.at[vb[k] + lanes], out_v.at[row0 + 10 + k],
                    sem_e).start()
            return 0
        lax.fori_loop(0, _CHUNK, samp, 0)

        def build2(t, _):
            f = t // (_CHUNK // 16)
            g = t - f * (_CHUNK // 16)
            n0 = g * 16
            gidx = spf_v[f, pl.ds(c * _CHUNK + n0, 16)] + f * _FIELD_SIZE
            idx2_v[f, pl.ds(n0, 16)] = gidx
            return 0
        lax.fori_loop(0, _NF * (_CHUNK // 16), build2, 0)

        def fire_lin(f, _):
            pltpu.make_async_copy(
                lin_hbm.at[idx2_v.at[f]], lin_v.at[f], sem_l).start()
            return 0
        lax.fori_loop(0, _NF, fire_lin, 0)

        def dense(g, _):
            n0 = g * 16
            for j in range(_ND):
                dvec = dx_v[j, pl.ds(c * _CHUNK + n0, 16)]
                erow = ew_v[j]
                for k in range(16):
                    out_v[(n0 + k) * _NSLOT + _NF + j] = erow * dvec[k]
            return 0
        lax.fori_loop(0, _CHUNK // 16, dense, 0)

        def drain_lin(f, _):
            pltpu.make_async_copy(
                lin_hbm.at[idx2_v.at[f]], lin_v.at[f], sem_l).wait()
            return 0
        lax.fori_loop(0, _NF, drain_lin, 0)

        def v1red(g, _):
            n0 = g * 16
            acc = jnp.zeros((16,), jnp.float32)
            for f in range(_NF):
                acc = acc + lin_v[f, pl.ds(n0, 16)]
            for j in range(_ND):
                acc = acc + dx_v[j, pl.ds(c * _CHUNK + n0, 16)] * lw_reg[j]
            v1_v[pl.ds(n0, 16)] = acc
            return 0
        lax.fori_loop(0, _CHUNK // 16, v1red, 0)

        pltpu.make_async_copy(
            v2_hbm.at[pl.ds(0, _NF * _CHUNK)],
            out_v.at[pl.ds(0, _NF * _CHUNK)], sem_e).wait()

        pltpu.make_async_copy(
            out_v, v2_hbm.at[pl.ds(base * _NSLOT, _CHUNK * _NSLOT)],
            sem_w).start()
        pltpu.make_async_copy(v1_v, v1_hbm.at[pl.ds(base, _CHUNK)],
                              sem_w).start()
        return 0

    lax.fori_loop(0, nchunk, chunk_body, 0)
    pltpu.make_async_copy(
        out_v, v2_hbm.at[pl.ds(0, _CHUNK * _NSLOT)], sem_w).wait()
    pltpu.make_async_copy(v1_v, v1_hbm.at[pl.ds(0, _CHUNK)], sem_w).wait()


def kernel(sparse_x, dense_x, lin_table, lin_w, emb_table, emb_w):
    n = sparse_x.shape[0]
    spw = n // _NW
    spf_b = sparse_x.reshape(_NW, spw, _NF).transpose(0, 2, 1)
    dx_b = dense_x.reshape(_NW, spw, _ND).transpose(0, 2, 1)
    lin_flat = lin_table.reshape(-1)
    emb_flat = emb_table.reshape(-1)
    lw = jnp.pad(lin_w.reshape(-1), (0, 16 - _ND))
    ew = emb_w.reshape(_ND, _D)

    mesh = plsc.VectorSubcoreMesh(core_axis_name="c", subcore_axis_name="s")
    run = functools.partial(
        pl.kernel,
        out_type=[
            jax.ShapeDtypeStruct((n,), jnp.float32),
            jax.ShapeDtypeStruct((n * _NSLOT, _D), jnp.float32),
        ],
        mesh=mesh,
        compiler_params=pltpu.CompilerParams(use_tc_tiling_on_sc=False),
        scratch_types=[
            pltpu.VMEM((spw, _NF), jnp.int32),
            pltpu.VMEM((_NF, spw), jnp.int32),
            pltpu.VMEM((_ND, spw), jnp.float32),
            pltpu.VMEM((_ND, _D), jnp.float32),
            pltpu.VMEM((16,), jnp.float32),
            pltpu.VMEM((_NF, _CHUNK), jnp.int32),
            pltpu.VMEM((_NF, _CHUNK), jnp.float32),
            pltpu.VMEM((_NSLOT * _CHUNK, _D), jnp.float32),
            pltpu.VMEM((_CHUNK,), jnp.float32),
            pltpu.SemaphoreType.DMA,
            pltpu.SemaphoreType.DMA,
            pltpu.SemaphoreType.DMA,
        ],
    )(_body)
    v1, v2 = run(sparse_x, spf_b, dx_b, lin_flat, lw, emb_flat, ew)
    return v1, v2.reshape(n, _NSLOT, _D)


# speedup vs baseline: 1.2374x; 1.2374x over previous
import functools

import jax
import jax.numpy as jnp
from jax import lax
from jax.experimental import pallas as pl
from jax.experimental.pallas import tpu as pltpu
from jax.experimental.pallas import tpu_sc as plsc

_NF = 26
_ND = 13
_D = 16
_NSLOT = _NF + _ND
_FIELD_SIZE = 40000
_NW = 32
_CHUNK = 128


def _body(sp_hbm, spf_hbm, dx_hbm, lin_hbm, lw_hbm, embf_hbm, ew_hbm,
          v1_hbm, v2_hbm,
          sp_v, spf_v, dx_v, ew_v, lw_v, idx2_v, lin_v, out_v, v1_v,
          sem_e, sem_l, sem_w):
    spw = sp_v.shape[0] // _NF
    nchunk = spw // _CHUNK
    wid = lax.axis_index("s") * 2 + lax.axis_index("c")

    pltpu.sync_copy(sp_hbm.at[pl.ds(wid * spw * _NF, spw * _NF)], sp_v)
    pltpu.sync_copy(spf_hbm.at[pl.ds(wid * spw * _NF, spw * _NF)], spf_v)
    pltpu.sync_copy(dx_hbm.at[pl.ds(wid * spw * _ND, spw * _ND)], dx_v)
    pltpu.sync_copy(ew_hbm, ew_v)
    pltpu.sync_copy(lw_hbm, lw_v)

    lanes = lax.iota(jnp.int32, 16)
    lw_reg = lw_v[pl.ds(0, 16)]
    off_lo = lanes * _FIELD_SIZE
    off_hi = (lanes + 10) * _FIELD_SIZE

    def chunk_body(c, _):
        base = wid * spw + c * _CHUNK

        @pl.when(c > 0)
        def _():
            pltpu.make_async_copy(
                out_v, v2_hbm.at[pl.ds(0, _CHUNK * _NSLOT * _D)], sem_w).wait()
            pltpu.make_async_copy(v1_v, v1_hbm.at[pl.ds(0, _CHUNK)], sem_w).wait()

        def samp(n, _):
            e0 = n * _NSLOT * _D
            s0 = (c * _CHUNK + n) * _NF
            va = (sp_v[pl.ds(s0, 16)] + off_lo) * _D
            vb = (sp_v[pl.ds(s0 + 10, 16)] + off_hi) * _D
            for k in range(16):
                pltpu.make_async_copy(
                    embf_hbm.at[va[k] + lanes],
                    out_v.at[pl.ds(e0 + k * _D, _D)], sem_e).start()
            for k in range(6, 16):
                pltpu.make_async_copy(
                    embf_hbm.at[vb[k] + lanes],
                    out_v.at[pl.ds(e0 + (10 + k) * _D, _D)], sem_e).start()
            return 0
        lax.fori_loop(0, _CHUNK, samp, 0)

        def build2(t, _):
            f = t // (_CHUNK // 16)
            g = t - f * (_CHUNK // 16)
            n0 = g * 16
            gidx = spf_v[pl.ds(f * spw + c * _CHUNK + n0, 16)] + f * _FIELD_SIZE
            idx2_v[f, pl.ds(n0, 16)] = gidx
            return 0
        lax.fori_loop(0, _NF * (_CHUNK // 16), build2, 0)

        def fire_lin(f, _):
            pltpu.make_async_copy(
                lin_hbm.at[idx2_v.at[f]], lin_v.at[f], sem_l).start()
            return 0
        lax.fori_loop(0, _NF, fire_lin, 0)

        def dense(g, _):
            n0 = g * 16
            for j in range(_ND):
                dvec = dx_v[pl.ds(j * spw + c * _CHUNK + n0, 16)]
                erow = ew_v[pl.ds(j * _D, 16)]
                for k in range(16):
                    out_v[pl.ds(((n0 + k) * _NSLOT + _NF + j) * _D, 16)] = (
                        erow * dvec[k])
            return 0
        lax.fori_loop(0, _CHUNK // 16, dense, 0)

        def drain_lin(f, _):
            pltpu.make_async_copy(
                lin_hbm.at[idx2_v.at[f]], lin_v.at[f], sem_l).wait()
            return 0
        lax.fori_loop(0, _NF, drain_lin, 0)

        def v1red(g, _):
            n0 = g * 16
            acc = jnp.zeros((16,), jnp.float32)
            for f in range(_NF):
                acc = acc + lin_v[f, pl.ds(n0, 16)]
            for j in range(_ND):
                acc = acc + dx_v[pl.ds(j * spw + c * _CHUNK + n0, 16)] * lw_reg[j]
            v1_v[pl.ds(n0, 16)] = acc
            return 0
        lax.fori_loop(0, _CHUNK // 16, v1red, 0)

        pltpu.make_async_copy(
            v2_hbm.at[pl.ds(0, _NF * _CHUNK * _D)],
            out_v.at[pl.ds(0, _NF * _CHUNK * _D)], sem_e).wait()

        pltpu.make_async_copy(
            out_v, v2_hbm.at[pl.ds(base * _NSLOT * _D, _CHUNK * _NSLOT * _D)],
            sem_w).start()
        pltpu.make_async_copy(v1_v, v1_hbm.at[pl.ds(base, _CHUNK)],
                              sem_w).start()
        return 0

    lax.fori_loop(0, nchunk, chunk_body, 0)
    pltpu.make_async_copy(
        out_v, v2_hbm.at[pl.ds(0, _CHUNK * _NSLOT * _D)], sem_w).wait()
    pltpu.make_async_copy(v1_v, v1_hbm.at[pl.ds(0, _CHUNK)], sem_w).wait()


def kernel(sparse_x, dense_x, lin_table, lin_w, emb_table, emb_w):
    n = sparse_x.shape[0]
    spw = n // _NW
    sp_flat = sparse_x.reshape(-1)
    spf_flat = sparse_x.reshape(_NW, spw, _NF).transpose(0, 2, 1).reshape(-1)
    dx_flat = dense_x.reshape(_NW, spw, _ND).transpose(0, 2, 1).reshape(-1)
    lin_flat = lin_table.reshape(-1)
    emb_flat = emb_table.reshape(-1)
    lw = jnp.pad(lin_w.reshape(-1), (0, 16 - _ND))
    ew = emb_w.reshape(-1)

    mesh = plsc.VectorSubcoreMesh(core_axis_name="c", subcore_axis_name="s")
    run = functools.partial(
        pl.kernel,
        out_type=[
            jax.ShapeDtypeStruct((n,), jnp.float32),
            jax.ShapeDtypeStruct((n * _NSLOT * _D,), jnp.float32),
        ],
        mesh=mesh,
        compiler_params=pltpu.CompilerParams(use_tc_tiling_on_sc=False),
        scratch_types=[
            pltpu.VMEM((spw * _NF,), jnp.int32),
            pltpu.VMEM((spw * _NF,), jnp.int32),
            pltpu.VMEM((spw * _ND,), jnp.float32),
            pltpu.VMEM((_ND * _D,), jnp.float32),
            pltpu.VMEM((16,), jnp.float32),
            pltpu.VMEM((_NF, _CHUNK), jnp.int32),
            pltpu.VMEM((_NF, _CHUNK), jnp.float32),
            pltpu.VMEM((_NSLOT * _CHUNK * _D,), jnp.float32),
            pltpu.VMEM((_CHUNK,), jnp.float32),
            pltpu.SemaphoreType.DMA,
            pltpu.SemaphoreType.DMA,
            pltpu.SemaphoreType.DMA,
        ],
    )(_body)
    v1, v2 = run(sp_flat, spf_flat, dx_flat, lin_flat, lw, emb_flat, ew)
    return v1, v2.reshape(n, _NSLOT, _D)


# speedup vs baseline: 4.0043x; 3.2362x over previous
import functools

import jax
import jax.numpy as jnp
from jax import lax
from jax.experimental import pallas as pl
from jax.experimental.pallas import tpu as pltpu
from jax.experimental.pallas import tpu_sc as plsc

_NF = 26
_ND = 13
_D = 16
_NSLOT = _NF + _ND
_FIELD_SIZE = 40000
_NW = 32
_CHUNK = 128


def _body(spf_hbm, dx_hbm, lin_hbm, lw_hbm, embp_hbm, ew_hbm,
          v1_hbm, v2_hbm,
          spf_v, dx_v, ew_v, lw_v, idx2_v, pb_v, lin_v, out_v, v1_v,
          sem_e, sem_l, sem_w):
    spw = spf_v.shape[0] // _NF
    nchunk = spw // _CHUNK
    wid = lax.axis_index("s") * 2 + lax.axis_index("c")
    vocab = _NF * _FIELD_SIZE
    emb_dhi = (vocab // 128) * 1024
    nbatch = spw * _NW
    v2_slot = (nbatch // 128) * 2 * 1024
    v2_dhi = (nbatch // 128) * 1024

    pltpu.sync_copy(spf_hbm.at[pl.ds(wid * spw * _NF, spw * _NF)], spf_v)
    pltpu.sync_copy(dx_hbm.at[pl.ds(wid * spw * _ND, spw * _ND)], dx_v)
    pltpu.sync_copy(ew_hbm, ew_v)
    pltpu.sync_copy(lw_hbm, lw_v)

    lanes = lax.iota(jnp.int32, 16)
    lw_reg = lw_v[pl.ds(0, 16)]

    def chunk_body(c, _):
        base = wid * spw + c * _CHUNK

        @pl.when(c > 0)
        def _():
            pltpu.make_async_copy(
                out_v, v2_hbm.at[pl.ds(0, _NSLOT * 2048)], sem_w).wait()
            pltpu.make_async_copy(v1_v, v1_hbm.at[pl.ds(0, _CHUNK)], sem_w).wait()

        def build(t, _):
            f = t // (_CHUNK // 16)
            g = t - f * (_CHUNK // 16)
            n0 = g * 16
            rvec = spf_v[pl.ds(f * spw + c * _CHUNK + n0, 16)] + f * _FIELD_SIZE
            idx2_v[f, pl.ds(n0, 16)] = rvec
            pvec = (rvec >> 7) * 1024 + (rvec & 127)
            o0 = f * 2048 + n0
            for d in range(_D):
                pltpu.make_async_copy(
                    embp_hbm.at[pvec + ((d >> 3) * emb_dhi + (d & 7) * 128)],
                    out_v.at[pl.ds(o0 + (d >> 3) * 1024 + (d & 7) * 128, 16)],
                    sem_e).start()
            return 0
        lax.fori_loop(0, _NF * (_CHUNK // 16), build, 0)

        def fire_lin(f, _):
            pltpu.make_async_copy(
                lin_hbm.at[idx2_v.at[f]], lin_v.at[f], sem_l).start()
            return 0
        lax.fori_loop(0, _NF, fire_lin, 0)

        def dense(t, _):
            j = t // (_CHUNK // 16)
            g = t - j * (_CHUNK // 16)
            n0 = g * 16
            dvec = dx_v[pl.ds(j * spw + c * _CHUNK + n0, 16)]
            erow = ew_v[pl.ds(j * _D, 16)]
            o0 = (_NF + j) * 2048 + n0
            for d in range(_D):
                out_v[pl.ds(o0 + (d >> 3) * 1024 + (d & 7) * 128, 16)] = (
                    dvec * erow[d])
            return 0
        lax.fori_loop(0, _ND * (_CHUNK // 16), dense, 0)

        def drain_lin(f, _):
            pltpu.make_async_copy(
                lin_hbm.at[idx2_v.at[f]], lin_v.at[f], sem_l).wait()
            return 0
        lax.fori_loop(0, _NF, drain_lin, 0)

        def v1red(g, _):
            n0 = g * 16
            acc = jnp.zeros((16,), jnp.float32)
            for f in range(_NF):
                acc = acc + lin_v[f, pl.ds(n0, 16)]
            for j in range(_ND):
                acc = acc + dx_v[pl.ds(j * spw + c * _CHUNK + n0, 16)] * lw_reg[j]
            v1_v[pl.ds(n0, 16)] = acc
            return 0
        lax.fori_loop(0, _CHUNK // 16, v1red, 0)

        pltpu.make_async_copy(
            v2_hbm.at[pl.ds(0, _NF * 2048)],
            out_v.at[pl.ds(0, _NF * 2048)], sem_e).wait()

        def wseg(s2, _):
            s = s2 >> 1
            dhi = s2 & 1
            pltpu.make_async_copy(
                out_v.at[pl.ds(s * 2048 + dhi * 1024, 1024)],
                v2_hbm.at[pl.ds(s * v2_slot + dhi * v2_dhi + base * 8, 1024)],
                sem_w).start()
            return 0
        lax.fori_loop(0, _NSLOT * 2, wseg, 0)
        pltpu.make_async_copy(v1_v, v1_hbm.at[pl.ds(base, _CHUNK)],
                              sem_w).start()
        return 0

    lax.fori_loop(0, nchunk, chunk_body, 0)
    pltpu.make_async_copy(
        out_v, v2_hbm.at[pl.ds(0, _NSLOT * 2048)], sem_w).wait()
    pltpu.make_async_copy(v1_v, v1_hbm.at[pl.ds(0, _CHUNK)], sem_w).wait()


def kernel(sparse_x, dense_x, lin_table, lin_w, emb_table, emb_w):
    n = sparse_x.shape[0]
    spw = n // _NW
    vocab = emb_table.shape[0]
    spf_flat = sparse_x.reshape(_NW, spw, _NF).transpose(0, 2, 1).reshape(-1)
    dx_flat = dense_x.reshape(_NW, spw, _ND).transpose(0, 2, 1).reshape(-1)
    lin_flat = lin_table.reshape(-1)
    lw = jnp.pad(lin_w.reshape(-1), (0, 16 - _ND))
    ew = emb_w.reshape(-1)
    emb_phys = (emb_table.T.reshape(2, 8, vocab // 128, 128)
                .transpose(0, 2, 1, 3).reshape(-1))

    mesh = plsc.VectorSubcoreMesh(core_axis_name="c", subcore_axis_name="s")
    run = functools.partial(
        pl.kernel,
        out_type=[
            jax.ShapeDtypeStruct((n,), jnp.float32),
            jax.ShapeDtypeStruct((n * _NSLOT * _D,), jnp.float32),
        ],
        mesh=mesh,
        compiler_params=pltpu.CompilerParams(use_tc_tiling_on_sc=False),
        scratch_types=[
            pltpu.VMEM((spw * _NF,), jnp.int32),
            pltpu.VMEM((spw * _ND,), jnp.float32),
            pltpu.VMEM((_ND * _D,), jnp.float32),
            pltpu.VMEM((16,), jnp.float32),
            pltpu.VMEM((_NF, _CHUNK), jnp.int32),
            pltpu.VMEM((_NF, _CHUNK), jnp.int32),
            pltpu.VMEM((_NF, _CHUNK), jnp.float32),
            pltpu.VMEM((_NSLOT * 2048,), jnp.float32),
            pltpu.VMEM((_CHUNK,), jnp.float32),
            pltpu.SemaphoreType.DMA,
            pltpu.SemaphoreType.DMA,
            pltpu.SemaphoreType.DMA,
        ],
    )(_body)
    v1, v2f = run(spf_flat, dx_flat, lin_flat, lw, emb_phys, ew)
    v2 = (v2f.reshape(_NSLOT, 2, n // 128, 8, 128)
          .transpose(2, 4, 0, 1, 3).reshape(n, _NSLOT, _D))
    return v1, v2
